# baseline (speedup 1.0000x reference)
import functools

import jax
import jax.numpy as jnp
from jax import lax
from jax.experimental import pallas as pl
from jax.experimental.pallas import tpu as pltpu
from jax.experimental.pallas import tpu_sc as plsc

D = 1024
NE = 64
NTOK = 2048
TM = 64
NPAD = 8192
NT = NPAD // TM
NW = 32
SEGS = 1
SEG = NPAD // SEGS
SEG_T = SEG // TM
GCH = 32
CCH = 32


def _router_body(x_ref, wr_ref, i1_ref, i2_ref, w1_ref, w2_ref):
    xb = x_ref[...]
    wr = wr_ref[...]
    logits = lax.dot_general(xb, wr, (((1,), (1,)), ((), ())),
                             preferred_element_type=jnp.float32)
    iota = lax.broadcasted_iota(jnp.int32, logits.shape, 1)
    m1 = jnp.max(logits, axis=1, keepdims=True)
    i1 = jnp.min(jnp.where(logits == m1, iota, NE), axis=1, keepdims=True)
    masked = jnp.where(iota == i1, -jnp.inf, logits)
    m2 = jnp.max(masked, axis=1, keepdims=True)
    i2 = jnp.min(jnp.where(masked == m2, iota, NE), axis=1, keepdims=True)
    w1 = 1.0 / (1.0 + jnp.exp(m2 - m1))
    i1_ref[...] = i1
    i2_ref[...] = i2
    w1_ref[...] = w1
    w2_ref[...] = 1.0 - w1


def _router(x2, Wr):
    return pl.pallas_call(
        _router_body,
        out_shape=[
            jax.ShapeDtypeStruct((NTOK, 1), jnp.int32),
            jax.ShapeDtypeStruct((NTOK, 1), jnp.int32),
            jax.ShapeDtypeStruct((NTOK, 1), jnp.float32),
            jax.ShapeDtypeStruct((NTOK, 1), jnp.float32),
        ],
    )(x2, Wr)


GNB = 3
GROWS = SEG // NW
GNCH = GROWS // GCH


def _gather_body(x_hbm, rt_hbm, out_hbm, idx_v, b0, b1, b2, g0, g1, g2,
                 s0, s1, s2):
    wid = lax.axis_index("s") * 2 + lax.axis_index("c")
    base = wid * GROWS
    bufs = (b0, b1, b2)
    gsem = (g0, g1, g2)
    wsem = (s0, s1, s2)
    pltpu.sync_copy(rt_hbm.at[pl.ds(base, GROWS)], idx_v)

    gd, wd = {}, {}

    def start_gather(c):
        b = c % GNB
        gd[c] = pltpu.async_copy(
            x_hbm.at[idx_v.at[pl.ds(c * GCH, GCH)]], bufs[b], gsem[b])

    def start_write(c):
        b = c % GNB
        wd[c] = pltpu.async_copy(
            bufs[b], out_hbm.at[pl.ds(base + c * GCH, GCH)], wsem[b])

    for c in range(GNCH):
        if c >= GNB:
            wd[c - GNB].wait()
        start_gather(c)
        if c >= 1:
            gd[c - 1].wait()
            start_write(c - 1)
    gd[GNCH - 1].wait()
    start_write(GNCH - 1)
    for c in range(max(GNCH - GNB, 0), GNCH):
        wd[c].wait()


def _gather(x2, row_token_seg):
    f = functools.partial(
        pl.kernel,
        mesh=plsc.VectorSubcoreMesh(core_axis_name="c", subcore_axis_name="s"),
        out_type=jax.ShapeDtypeStruct((SEG, D // 2), jnp.int32),
        scratch_types=[
            pltpu.VMEM((GROWS,), jnp.int32),
            pltpu.VMEM((GCH, D // 2), jnp.int32),
            pltpu.VMEM((GCH, D // 2), jnp.int32),
            pltpu.VMEM((GCH, D // 2), jnp.int32),
            pltpu.SemaphoreType.DMA,
            pltpu.SemaphoreType.DMA,
            pltpu.SemaphoreType.DMA,
            pltpu.SemaphoreType.DMA,
            pltpu.SemaphoreType.DMA,
            pltpu.SemaphoreType.DMA,
        ],
    )(_gather_body)
    return f(x2, row_token_seg)


_RSQRT2 = 0.7071067811865476


def _ffn_body(te_ref, nt_ref, xs_ref, w1_ref, b1_ref, w2_ref, b2_ref,
              ysin_ref, ys_ref):
    j = pl.program_id(0)

    @pl.when(j < nt_ref[0])
    def _():
        xb = xs_ref[...].astype(jnp.float32)
        h = lax.dot_general(xb, w1_ref[0], (((1,), (1,)), ((), ())),
                            preferred_element_type=jnp.float32)
        h = h + b1_ref[0]
        h = 0.5 * h * (1.0 + lax.erf(h * _RSQRT2))
        y = lax.dot_general(h, w2_ref[0], (((1,), (1,)), ((), ())),
                            preferred_element_type=jnp.float32)
        ys_ref[...] = y + b2_ref[0]


def _ffn_seg(s, tile_expert_s, nact_s, xs_s, W1, b1r, W2, b2r, ys_in):
    def _jm(j, nt):
        return jnp.maximum(jnp.minimum(j, nt[0] - 1), 0)

    grid_spec = pltpu.PrefetchScalarGridSpec(
        num_scalar_prefetch=2,
        grid=(SEG_T,),
        in_specs=[
            pl.BlockSpec((TM, D), lambda j, te, nt: (_jm(j, nt), 0)),
            pl.BlockSpec((1, D, D),
                         lambda j, te, nt: (te[_jm(j, nt)], 0, 0)),
            pl.BlockSpec((1, 1, D),
                         lambda j, te, nt: (te[_jm(j, nt)], 0, 0)),
            pl.BlockSpec((1, D, D),
                         lambda j, te, nt: (te[_jm(j, nt)], 0, 0)),
            pl.BlockSpec((1, 1, D),
                         lambda j, te, nt: (te[_jm(j, nt)], 0, 0)),
            pl.BlockSpec((TM, D),
                         lambda j, te, nt: (s * SEG_T + _jm(j, nt), 0)),
        ],
        out_specs=pl.BlockSpec((TM, D),
                               lambda j, te, nt: (s * SEG_T + _jm(j, nt), 0)),
    )
    return pl.pallas_call(
        _ffn_body,
        grid_spec=grid_spec,
        out_shape=jax.ShapeDtypeStruct((NPAD, D), jnp.float32),
        input_output_aliases={7: 0},
        compiler_params=pltpu.CompilerParams(
            dimension_semantics=("arbitrary",)),
    )(tile_expert_s, nact_s, xs_s, W1, b1r, W2, b2r, ys_in)


def _combine_body(ys_hbm, pa_hbm, pb_hbm, wa_hbm, wb_hbm, out_hbm,
                  ia_v, ib_v, wa_v, wb_v, ba_v, bb_v, sa, sb):
    wid = lax.axis_index("s") * 2 + lax.axis_index("c")
    base = wid * (NTOK // NW)

    def chunk(c, carry):
        off = base + c * CCH
        pltpu.sync_copy(pa_hbm.at[pl.ds(off, CCH)], ia_v)
        pltpu.sync_copy(pb_hbm.at[pl.ds(off, CCH)], ib_v)
        pltpu.sync_copy(wa_hbm.at[pl.ds(off, CCH)], wa_v)
        pltpu.sync_copy(wb_hbm.at[pl.ds(off, CCH)], wb_v)
        cpa = pltpu.async_copy(ys_hbm.at[ia_v], ba_v, sa)
        cpb = pltpu.async_copy(ys_hbm.at[ib_v], bb_v, sb)
        cpa.wait()
        cpb.wait()

        def row(r, carry2):
            wa = wa_v[r, :]
            wb = wb_v[r, :]
            for i in range(D // 16):
                sl = pl.ds(i * 16, 16)
                ba_v[r, sl] = ba_v[r, sl] * wa + bb_v[r, sl] * wb
            return carry2

        lax.fori_loop(0, CCH, row, 0)
        pltpu.sync_copy(ba_v, out_hbm.at[pl.ds(off, CCH)])
        return carry

    lax.fori_loop(0, (NTOK // NW) // CCH, chunk, 0)


def _combine(ys, posA, posB, wA, wB):
    f = functools.partial(
        pl.kernel,
        mesh=plsc.VectorSubcoreMesh(core_axis_name="c", subcore_axis_name="s"),
        out_type=jax.ShapeDtypeStruct((NTOK, D), jnp.float32),
        scratch_types=[
            pltpu.VMEM((CCH,), jnp.int32),
            pltpu.VMEM((CCH,), jnp.int32),
            pltpu.VMEM((CCH, 16), jnp.float32),
            pltpu.VMEM((CCH, 16), jnp.float32),
            pltpu.VMEM((CCH, D), jnp.float32),
            pltpu.VMEM((CCH, D), jnp.float32),
            pltpu.SemaphoreType.DMA,
            pltpu.SemaphoreType.DMA,
        ],
    )(_combine_body)
    return f(ys, posA, posB, wA, wB)


def _dispatch_plan(i1, i2):
    e = jnp.concatenate([i1, i2])
    oh = (e[:, None] == jnp.arange(NE, dtype=jnp.int32)[None, :])
    cum = jnp.cumsum(oh.astype(jnp.int32), axis=0)
    counts = cum[-1]
    rank = jnp.take_along_axis(cum, e[:, None], axis=1)[:, 0] - 1
    padded = ((counts + TM - 1) // TM) * TM
    pad_end = jnp.cumsum(padded)
    pad_off = pad_end - padded
    pos = pad_off[e] + rank

    token_ids = jnp.tile(jnp.arange(NTOK, dtype=jnp.int32), 2)
    row_token = jnp.zeros((NPAD,), jnp.int32).at[pos].set(token_ids)

    total = pad_end[-1]
    nactive = (total // TM).astype(jnp.int32)
    tiles = jnp.arange(NT, dtype=jnp.int32) * TM
    raw = jnp.minimum(
        jnp.searchsorted(pad_end, tiles, side="right").astype(jnp.int32),
        NE - 1)
    last = raw[jnp.maximum(nactive - 1, 0)]
    tile_expert = jnp.where(tiles < total, raw, last)
    return pos[:NTOK], pos[NTOK:], row_token, tile_expert, nactive


def kernel(x, Wr, W1, b1, W2, b2):
    Bx, L, Dx = x.shape
    x2 = x.reshape(L, Dx)
    i1, i2, w1, w2 = _router(x2, Wr)
    i1, i2 = i1[:, 0], i2[:, 0]
    wA, wB = w1[:, 0], w2[:, 0]
    posA, posB, row_token, tile_expert, nactive = _dispatch_plan(i1, i2)

    x2i = lax.bitcast_convert_type(
        x2.astype(jnp.bfloat16).reshape(NTOK, D // 2, 2), jnp.int32)
    xs32 = [_gather(x2i, row_token[s * SEG:(s + 1) * SEG])
            for s in range(SEGS)]
    xs = [lax.bitcast_convert_type(a, jnp.bfloat16).reshape(SEG, D)
          for a in xs32]

    b1r = b1.reshape(NE, 1, D)
    b2r = b2.reshape(NE, 1, D)
    ys = jnp.zeros((NPAD, D), jnp.float32)
    for s in range(SEGS):
        te_s = tile_expert[s * SEG_T:(s + 1) * SEG_T]
        na_s = jnp.clip(nactive - s * SEG_T, 0, SEG_T).reshape(1)
        ys = _ffn_seg(s, te_s, na_s, xs[s], W1, b1r, W2, b2r, ys)

    wA16 = jnp.broadcast_to(wA[:, None], (NTOK, 16))
    wB16 = jnp.broadcast_to(wB[:, None], (NTOK, 16))
    out = _combine(ys, posA, posB, wA16, wB16)
    return out.reshape(Bx, L, Dx)


# speedup vs baseline: 1.2336x; 1.2336x over previous
import functools

import jax
import jax.numpy as jnp
from jax import lax
from jax.experimental import pallas as pl
from jax.experimental.pallas import tpu as pltpu
from jax.experimental.pallas import tpu_sc as plsc

D = 1024
NE = 64
NTOK = 2048
TM = 64
NPAD = 8192
NT = NPAD // TM
NW = 32
SEGS = 1
SEG = NPAD // SEGS
SEG_T = SEG // TM
GCH = 32
CCH = 32


def _router_body(x_ref, wr_ref, i1_ref, i2_ref, w1_ref, w2_ref):
    xb = x_ref[...]
    wr = wr_ref[...]
    logits = lax.dot_general(xb, wr, (((1,), (1,)), ((), ())),
                             preferred_element_type=jnp.float32)
    iota = lax.broadcasted_iota(jnp.int32, logits.shape, 1)
    m1 = jnp.max(logits, axis=1, keepdims=True)
    i1 = jnp.min(jnp.where(logits == m1, iota, NE), axis=1, keepdims=True)
    masked = jnp.where(iota == i1, -jnp.inf, logits)
    m2 = jnp.max(masked, axis=1, keepdims=True)
    i2 = jnp.min(jnp.where(masked == m2, iota, NE), axis=1, keepdims=True)
    w1 = 1.0 / (1.0 + jnp.exp(m2 - m1))
    i1_ref[...] = i1
    i2_ref[...] = i2
    w1_ref[...] = w1
    w2_ref[...] = 1.0 - w1


def _router(x2, Wr):
    return pl.pallas_call(
        _router_body,
        out_shape=[
            jax.ShapeDtypeStruct((NTOK, 1), jnp.int32),
            jax.ShapeDtypeStruct((NTOK, 1), jnp.int32),
            jax.ShapeDtypeStruct((NTOK, 1), jnp.float32),
            jax.ShapeDtypeStruct((NTOK, 1), jnp.float32),
        ],
    )(x2, Wr)


GNB = 3
GROWS = SEG // NW
GNCH = GROWS // GCH


def _gather_body(x_hbm, rt_hbm, out_hbm, idx_v, b0, b1, b2, g0, g1, g2,
                 s0, s1, s2):
    wid = lax.axis_index("s") * 2 + lax.axis_index("c")
    base = wid * GROWS
    bufs = (b0, b1, b2)
    gsem = (g0, g1, g2)
    wsem = (s0, s1, s2)
    pltpu.sync_copy(rt_hbm.at[pl.ds(base, GROWS)], idx_v)

    gd, wd = {}, {}

    def start_gather(c):
        b = c % GNB
        gd[c] = pltpu.async_copy(
            x_hbm.at[idx_v.at[pl.ds(c * GCH, GCH)]], bufs[b], gsem[b])

    def start_write(c):
        b = c % GNB
        wd[c] = pltpu.async_copy(
            bufs[b], out_hbm.at[pl.ds(base + c * GCH, GCH)], wsem[b])

    for c in range(GNCH):
        if c >= GNB:
            wd[c - GNB].wait()
        start_gather(c)
        if c >= 1:
            gd[c - 1].wait()
            start_write(c - 1)
    gd[GNCH - 1].wait()
    start_write(GNCH - 1)
    for c in range(max(GNCH - GNB, 0), GNCH):
        wd[c].wait()


def _gather(x2, row_token_seg):
    f = functools.partial(
        pl.kernel,
        mesh=plsc.VectorSubcoreMesh(core_axis_name="c", subcore_axis_name="s"),
        out_type=jax.ShapeDtypeStruct((SEG, D), jnp.float32),
        scratch_types=[
            pltpu.VMEM((GROWS,), jnp.int32),
            pltpu.VMEM((GCH, D), jnp.float32),
            pltpu.VMEM((GCH, D), jnp.float32),
            pltpu.VMEM((GCH, D), jnp.float32),
            pltpu.SemaphoreType.DMA,
            pltpu.SemaphoreType.DMA,
            pltpu.SemaphoreType.DMA,
            pltpu.SemaphoreType.DMA,
            pltpu.SemaphoreType.DMA,
            pltpu.SemaphoreType.DMA,
        ],
    )(_gather_body)
    return f(x2, row_token_seg)


_RSQRT2 = 0.7071067811865476


def _ffn_body(te_ref, nt_ref, xs_ref, w1_ref, b1_ref, w2_ref, b2_ref,
              ysin_ref, ys_ref):
    j = pl.program_id(0)

    @pl.when(j < nt_ref[0])
    def _():
        xb = xs_ref[...]
        h = lax.dot_general(xb, w1_ref[0], (((1,), (1,)), ((), ())),
                            preferred_element_type=jnp.float32)
        h = h + b1_ref[0]
        h = 0.5 * h * (1.0 + lax.erf(h * _RSQRT2))
        y = lax.dot_general(h, w2_ref[0], (((1,), (1,)), ((), ())),
                            preferred_element_type=jnp.float32)
        ys_ref[...] = y + b2_ref[0]


def _ffn_seg(s, tile_expert_s, nact_s, xs_s, W1, b1r, W2, b2r, ys_in):
    def _jm(j, nt):
        return jnp.maximum(jnp.minimum(j, nt[0] - 1), 0)

    grid_spec = pltpu.PrefetchScalarGridSpec(
        num_scalar_prefetch=2,
        grid=(SEG_T,),
        in_specs=[
            pl.BlockSpec((TM, D), lambda j, te, nt: (_jm(j, nt), 0)),
            pl.BlockSpec((1, D, D),
                         lambda j, te, nt: (te[_jm(j, nt)], 0, 0)),
            pl.BlockSpec((1, 1, D),
                         lambda j, te, nt: (te[_jm(j, nt)], 0, 0)),
            pl.BlockSpec((1, D, D),
                         lambda j, te, nt: (te[_jm(j, nt)], 0, 0)),
            pl.BlockSpec((1, 1, D),
                         lambda j, te, nt: (te[_jm(j, nt)], 0, 0)),
            pl.BlockSpec((TM, D),
                         lambda j, te, nt: (s * SEG_T + _jm(j, nt), 0)),
        ],
        out_specs=pl.BlockSpec((TM, D),
                               lambda j, te, nt: (s * SEG_T + _jm(j, nt), 0)),
    )
    return pl.pallas_call(
        _ffn_body,
        grid_spec=grid_spec,
        out_shape=jax.ShapeDtypeStruct((NPAD, D), jnp.float32),
        input_output_aliases={7: 0},
        compiler_params=pltpu.CompilerParams(
            dimension_semantics=("arbitrary",)),
    )(tile_expert_s, nact_s, xs_s, W1, b1r, W2, b2r, ys_in)


def _combine_body(ys_hbm, pa_hbm, pb_hbm, wa_hbm, wb_hbm, out_hbm,
                  ia_v, ib_v, wa_v, wb_v, ba_v, bb_v, sa, sb):
    wid = lax.axis_index("s") * 2 + lax.axis_index("c")
    base = wid * (NTOK // NW)

    def chunk(c, carry):
        off = base + c * CCH
        pltpu.sync_copy(pa_hbm.at[pl.ds(off, CCH)], ia_v)
        pltpu.sync_copy(pb_hbm.at[pl.ds(off, CCH)], ib_v)
        pltpu.sync_copy(wa_hbm.at[pl.ds(off, CCH)], wa_v)
        pltpu.sync_copy(wb_hbm.at[pl.ds(off, CCH)], wb_v)
        cpa = pltpu.async_copy(ys_hbm.at[ia_v], ba_v, sa)
        cpb = pltpu.async_copy(ys_hbm.at[ib_v], bb_v, sb)
        cpa.wait()
        cpb.wait()

        def row(r, carry2):
            wa = wa_v[r, :]
            wb = wb_v[r, :]
            for i in range(D // 16):
                sl = pl.ds(i * 16, 16)
                ba_v[r, sl] = ba_v[r, sl] * wa + bb_v[r, sl] * wb
            return carry2

        lax.fori_loop(0, CCH, row, 0)
        pltpu.sync_copy(ba_v, out_hbm.at[pl.ds(off, CCH)])
        return carry

    lax.fori_loop(0, (NTOK // NW) // CCH, chunk, 0)


def _combine(ys, posA, posB, wA, wB):
    f = functools.partial(
        pl.kernel,
        mesh=plsc.VectorSubcoreMesh(core_axis_name="c", subcore_axis_name="s"),
        out_type=jax.ShapeDtypeStruct((NTOK, D), jnp.float32),
        scratch_types=[
            pltpu.VMEM((CCH,), jnp.int32),
            pltpu.VMEM((CCH,), jnp.int32),
            pltpu.VMEM((CCH, 16), jnp.float32),
            pltpu.VMEM((CCH, 16), jnp.float32),
            pltpu.VMEM((CCH, D), jnp.float32),
            pltpu.VMEM((CCH, D), jnp.float32),
            pltpu.SemaphoreType.DMA,
            pltpu.SemaphoreType.DMA,
        ],
    )(_combine_body)
    return f(ys, posA, posB, wA, wB)


def _dispatch_plan(i1, i2):
    e = jnp.concatenate([i1, i2])
    oh = (e[:, None] == jnp.arange(NE, dtype=jnp.int32)[None, :])
    cum = jnp.cumsum(oh.astype(jnp.int32), axis=0)
    counts = cum[-1]
    rank = jnp.take_along_axis(cum, e[:, None], axis=1)[:, 0] - 1
    padded = ((counts + TM - 1) // TM) * TM
    pad_end = jnp.cumsum(padded)
    pad_off = pad_end - padded
    pos = pad_off[e] + rank

    token_ids = jnp.tile(jnp.arange(NTOK, dtype=jnp.int32), 2)
    row_token = jnp.zeros((NPAD,), jnp.int32).at[pos].set(token_ids)

    total = pad_end[-1]
    nactive = (total // TM).astype(jnp.int32)
    tiles = jnp.arange(NT, dtype=jnp.int32) * TM
    raw = jnp.minimum(
        jnp.searchsorted(pad_end, tiles, side="right").astype(jnp.int32),
        NE - 1)
    last = raw[jnp.maximum(nactive - 1, 0)]
    tile_expert = jnp.where(tiles < total, raw, last)
    return pos[:NTOK], pos[NTOK:], row_token, tile_expert, nactive


def kernel(x, Wr, W1, b1, W2, b2):
    Bx, L, Dx = x.shape
    x2 = x.reshape(L, Dx)
    i1, i2, w1, w2 = _router(x2, Wr)
    i1, i2 = i1[:, 0], i2[:, 0]
    wA, wB = w1[:, 0], w2[:, 0]
    posA, posB, row_token, tile_expert, nactive = _dispatch_plan(i1, i2)

    xs = [_gather(x2, row_token[s * SEG:(s + 1) * SEG]) for s in range(SEGS)]

    b1r = b1.reshape(NE, 1, D)
    b2r = b2.reshape(NE, 1, D)
    ys = jnp.zeros((NPAD, D), jnp.float32)
    for s in range(SEGS):
        te_s = tile_expert[s * SEG_T:(s + 1) * SEG_T]
        na_s = jnp.clip(nactive - s * SEG_T, 0, SEG_T).reshape(1)
        ys = _ffn_seg(s, te_s, na_s, xs[s], W1, b1r, W2, b2r, ys)

    wA16 = jnp.broadcast_to(wA[:, None], (NTOK, 16))
    wB16 = jnp.broadcast_to(wB[:, None], (NTOK, 16))
    out = _combine(ys, posA, posB, wA16, wB16)
    return out.reshape(Bx, L, Dx)


# speedup vs baseline: 1.2718x; 1.0310x over previous
import functools

import jax
import jax.numpy as jnp
from jax import lax
from jax.experimental import pallas as pl
from jax.experimental.pallas import tpu as pltpu
from jax.experimental.pallas import tpu_sc as plsc

D = 1024
NE = 64
NTOK = 2048
TM = 64
NPAD = 8192
NT = NPAD // TM
NW = 32
SEGS = 1
SEG = NPAD // SEGS
SEG_T = SEG // TM
GCH = 32
CCH = 32


def _router_body(x_ref, wr_ref, i1_ref, i2_ref, w1_ref, w2_ref):
    xb = x_ref[...]
    wr = wr_ref[...]
    logits = lax.dot_general(xb, wr, (((1,), (1,)), ((), ())),
                             preferred_element_type=jnp.float32)
    iota = lax.broadcasted_iota(jnp.int32, logits.shape, 1)
    m1 = jnp.max(logits, axis=1, keepdims=True)
    i1 = jnp.min(jnp.where(logits == m1, iota, NE), axis=1, keepdims=True)
    masked = jnp.where(iota == i1, -jnp.inf, logits)
    m2 = jnp.max(masked, axis=1, keepdims=True)
    i2 = jnp.min(jnp.where(masked == m2, iota, NE), axis=1, keepdims=True)
    w1 = 1.0 / (1.0 + jnp.exp(m2 - m1))
    i1_ref[...] = i1
    i2_ref[...] = i2
    w1_ref[...] = w1
    w2_ref[...] = 1.0 - w1


def _router(x2, Wr):
    return pl.pallas_call(
        _router_body,
        out_shape=[
            jax.ShapeDtypeStruct((NTOK, 1), jnp.int32),
            jax.ShapeDtypeStruct((NTOK, 1), jnp.int32),
            jax.ShapeDtypeStruct((NTOK, 1), jnp.float32),
            jax.ShapeDtypeStruct((NTOK, 1), jnp.float32),
        ],
    )(x2, Wr)


GNB = 3
GROWS = SEG // NW
GNCH = GROWS // GCH


def _gather_body(x_hbm, rt_hbm, out_hbm, idx_v, b0, b1, b2, g0, g1, g2,
                 s0, s1, s2):
    wid = lax.axis_index("s") * 2 + lax.axis_index("c")
    base = wid * GROWS
    bufs = (b0, b1, b2)
    gsem = (g0, g1, g2)
    wsem = (s0, s1, s2)
    pltpu.sync_copy(rt_hbm.at[pl.ds(base, GROWS)], idx_v)

    gd, wd = {}, {}

    def start_gather(c):
        b = c % GNB
        gd[c] = pltpu.async_copy(
            x_hbm.at[idx_v.at[pl.ds(c * GCH, GCH)]], bufs[b], gsem[b])

    def start_write(c):
        b = c % GNB
        wd[c] = pltpu.async_copy(
            bufs[b], out_hbm.at[pl.ds(base + c * GCH, GCH)], wsem[b])

    for c in range(GNCH):
        if c >= GNB:
            wd[c - GNB].wait()
        start_gather(c)
        if c >= 1:
            gd[c - 1].wait()
            start_write(c - 1)
    gd[GNCH - 1].wait()
    start_write(GNCH - 1)
    for c in range(max(GNCH - GNB, 0), GNCH):
        wd[c].wait()


def _gather(x2, row_token_seg):
    f = functools.partial(
        pl.kernel,
        mesh=plsc.VectorSubcoreMesh(core_axis_name="c", subcore_axis_name="s"),
        out_type=jax.ShapeDtypeStruct((SEG, D), jnp.float32),
        scratch_types=[
            pltpu.VMEM((GROWS,), jnp.int32),
            pltpu.VMEM((GCH, D), jnp.float32),
            pltpu.VMEM((GCH, D), jnp.float32),
            pltpu.VMEM((GCH, D), jnp.float32),
            pltpu.SemaphoreType.DMA,
            pltpu.SemaphoreType.DMA,
            pltpu.SemaphoreType.DMA,
            pltpu.SemaphoreType.DMA,
            pltpu.SemaphoreType.DMA,
            pltpu.SemaphoreType.DMA,
        ],
    )(_gather_body)
    return f(x2, row_token_seg)


_RSQRT2 = 0.7071067811865476


def _ffn_body(te_ref, nt_ref, xs_ref, w1_ref, b1_ref, w2_ref, b2_ref,
              ys_ref):
    j = pl.program_id(0)

    @pl.when(j < nt_ref[0])
    def _():
        xb = xs_ref[...]
        h = lax.dot_general(xb, w1_ref[0], (((1,), (1,)), ((), ())),
                            preferred_element_type=jnp.float32)
        h = h + b1_ref[0]
        h = 0.5 * h * (1.0 + lax.erf(h * _RSQRT2))
        y = lax.dot_general(h, w2_ref[0], (((1,), (1,)), ((), ())),
                            preferred_element_type=jnp.float32)
        ys_ref[...] = y + b2_ref[0]


def _ffn(tile_expert, nact, xs_full, W1, b1r, W2, b2r):
    def _jm(j, nt):
        return jnp.maximum(jnp.minimum(j, nt[0] - 1), 0)

    grid_spec = pltpu.PrefetchScalarGridSpec(
        num_scalar_prefetch=2,
        grid=(NT,),
        in_specs=[
            pl.BlockSpec((TM, D), lambda j, te, nt: (_jm(j, nt), 0)),
            pl.BlockSpec((1, D, D),
                         lambda j, te, nt: (te[_jm(j, nt)], 0, 0)),
            pl.BlockSpec((1, 1, D),
                         lambda j, te, nt: (te[_jm(j, nt)], 0, 0)),
            pl.BlockSpec((1, D, D),
                         lambda j, te, nt: (te[_jm(j, nt)], 0, 0)),
            pl.BlockSpec((1, 1, D),
                         lambda j, te, nt: (te[_jm(j, nt)], 0, 0)),
        ],
        out_specs=pl.BlockSpec((TM, D), lambda j, te, nt: (_jm(j, nt), 0)),
    )
    return pl.pallas_call(
        _ffn_body,
        grid_spec=grid_spec,
        out_shape=jax.ShapeDtypeStruct((NPAD, D), jnp.float32),
        compiler_params=pltpu.CompilerParams(
            dimension_semantics=("arbitrary",)),
    )(tile_expert, nact, xs_full, W1, b1r, W2, b2r)


def _combine_body(ys_hbm, pa_hbm, pb_hbm, wa_hbm, out_hbm,
                  ia_v, ib_v, wa_v, ba_v, bb_v, sa, sb):
    wid = lax.axis_index("s") * 2 + lax.axis_index("c")
    base = wid * (NTOK // NW)

    def chunk(c, carry):
        off = base + c * CCH
        pltpu.sync_copy(pa_hbm.at[pl.ds(off, CCH)], ia_v)
        pltpu.sync_copy(pb_hbm.at[pl.ds(off, CCH)], ib_v)
        pltpu.sync_copy(wa_hbm.at[pl.ds(off, CCH)], wa_v)
        cpa = pltpu.async_copy(ys_hbm.at[ia_v], ba_v, sa)
        cpb = pltpu.async_copy(ys_hbm.at[ib_v], bb_v, sb)
        cpa.wait()
        cpb.wait()

        def row(r, carry2):
            wa = wa_v[r, :]
            for i in range(D // 16):
                sl = pl.ds(i * 16, 16)
                b = bb_v[r, sl]
                ba_v[r, sl] = b + (ba_v[r, sl] - b) * wa
            return carry2

        lax.fori_loop(0, CCH, row, 0)
        pltpu.sync_copy(ba_v, out_hbm.at[pl.ds(off, CCH)])
        return carry

    lax.fori_loop(0, (NTOK // NW) // CCH, chunk, 0)


def _combine(ys, posA, posB, wA):
    f = functools.partial(
        pl.kernel,
        mesh=plsc.VectorSubcoreMesh(core_axis_name="c", subcore_axis_name="s"),
        out_type=jax.ShapeDtypeStruct((NTOK, D), jnp.float32),
        scratch_types=[
            pltpu.VMEM((CCH,), jnp.int32),
            pltpu.VMEM((CCH,), jnp.int32),
            pltpu.VMEM((CCH, 16), jnp.float32),
            pltpu.VMEM((CCH, D), jnp.float32),
            pltpu.VMEM((CCH, D), jnp.float32),
            pltpu.SemaphoreType.DMA,
            pltpu.SemaphoreType.DMA,
        ],
    )(_combine_body)
    return f(ys, posA, posB, wA)


def _dispatch_plan(i1, i2):
    e = jnp.concatenate([i1, i2])
    oh = (e[:, None] == jnp.arange(NE, dtype=jnp.int32)[None, :])
    cum = jnp.cumsum(oh.astype(jnp.int32), axis=0)
    counts = cum[-1]
    rank = jnp.take_along_axis(cum, e[:, None], axis=1)[:, 0] - 1
    padded = ((counts + TM - 1) // TM) * TM
    pad_end = jnp.cumsum(padded)
    pad_off = pad_end - padded
    pos = pad_off[e] + rank

    token_ids = jnp.tile(jnp.arange(NTOK, dtype=jnp.int32), 2)
    row_token = jnp.zeros((NPAD,), jnp.int32).at[pos].set(token_ids)

    total = pad_end[-1]
    nactive = (total // TM).astype(jnp.int32)
    tiles = jnp.arange(NT, dtype=jnp.int32) * TM
    raw = jnp.minimum(
        jnp.searchsorted(pad_end, tiles, side="right").astype(jnp.int32),
        NE - 1)
    last = raw[jnp.maximum(nactive - 1, 0)]
    tile_expert = jnp.where(tiles < total, raw, last)
    return pos[:NTOK], pos[NTOK:], row_token, tile_expert, nactive


def kernel(x, Wr, W1, b1, W2, b2):
    Bx, L, Dx = x.shape
    x2 = x.reshape(L, Dx)
    i1, i2, w1, w2 = _router(x2, Wr)
    i1, i2 = i1[:, 0], i2[:, 0]
    wA = w1[:, 0]
    posA, posB, row_token, tile_expert, nactive = _dispatch_plan(i1, i2)

    xs = _gather(x2, row_token)
    b1r = b1.reshape(NE, 1, D)
    b2r = b2.reshape(NE, 1, D)
    ys = _ffn(tile_expert, nactive.reshape(1), xs, W1, b1r, W2, b2r)

    wA16 = jnp.broadcast_to(wA[:, None], (NTOK, 16))
    out = _combine(ys, posA, posB, wA16)
    return out.reshape(Bx, L, Dx)


# speedup vs baseline: 1.4181x; 1.1150x over previous
import functools

import jax
import jax.numpy as jnp
from jax import lax
from jax.experimental import pallas as pl
from jax.experimental.pallas import tpu as pltpu
from jax.experimental.pallas import tpu_sc as plsc

D = 1024
NE = 64
NTOK = 2048
TM = 64
NPAD = 8192
NT = NPAD // TM
NW = 32
SEGS = 1
SEG = NPAD // SEGS
SEG_T = SEG // TM
GCH = 32
CCH = 32


def _router_body(x_ref, wr_ref, wa_ref, pos_ref, te_ref, na_ref,
                 ohf_ref, rank_ref):
    xb = x_ref[...]
    wr = wr_ref[...]
    logits = lax.dot_general(xb, wr, (((1,), (1,)), ((), ())),
                             preferred_element_type=jnp.float32)
    iota = lax.broadcasted_iota(jnp.int32, logits.shape, 1)
    m1 = jnp.max(logits, axis=1, keepdims=True)
    i1 = jnp.min(jnp.where(logits == m1, iota, NE), axis=1, keepdims=True)
    masked = jnp.where(iota == i1, -jnp.inf, logits)
    m2 = jnp.max(masked, axis=1, keepdims=True)
    i2 = jnp.min(jnp.where(masked == m2, iota, NE), axis=1, keepdims=True)
    wa_ref[...] = 1.0 / (1.0 + jnp.exp(m2 - m1))

    ohf_ref[0:NTOK, :] = (iota == i1).astype(jnp.float32)
    ohf_ref[NTOK:2 * NTOK, :] = (iota == i2).astype(jnp.float32)

    rl = lax.broadcasted_iota(jnp.int32, (128, 128), 0)
    cl = lax.broadcasted_iota(jnp.int32, (128, 128), 1)
    L = (rl >= cl).astype(jnp.float32)

    def chunk(ci, carry):
        blk = ohf_ref[pl.ds(ci * 128, 128), :]
        incl = lax.dot_general(L, blk, (((1,), (0,)), ((), ())),
                               preferred_element_type=jnp.float32)
        cum = incl + carry
        rank_ref[pl.ds(ci * 128, 128), :] = (
            jnp.sum(cum * blk, axis=1, keepdims=True) - 1.0)
        return carry + incl[127:128, :]

    counts = lax.fori_loop(0, (2 * NTOK) // 128, chunk,
                           jnp.zeros((1, NE), jnp.float32))

    cnt_i = counts.astype(jnp.int32)
    padded = (((cnt_i + TM - 1) // TM) * TM).astype(jnp.float32)
    r64 = lax.broadcasted_iota(jnp.int32, (NE, NE), 0)
    c64 = lax.broadcasted_iota(jnp.int32, (NE, NE), 1)
    U = (r64 <= c64).astype(jnp.float32)
    pad_end = lax.dot_general(padded, U, (((1,), (0,)), ((), ())),
                              preferred_element_type=jnp.float32)
    pad_off = pad_end - padded

    posf = jnp.sum(ohf_ref[...] * pad_off, axis=1, keepdims=True)
    pos_ref[...] = (posf + rank_ref[...]).astype(jnp.int32)

    total = pad_end[0:1, NE - 1:NE]
    na_ref[...] = (total.astype(jnp.int32)) // TM

    tio = (lax.broadcasted_iota(jnp.int32, (NT, NE), 0) * TM
           ).astype(jnp.float32)
    raw = jnp.minimum(
        jnp.sum((pad_end <= tio).astype(jnp.float32), axis=1, keepdims=True),
        float(NE - 1))
    lastte = jnp.sum((pad_end <= total - TM).astype(jnp.float32),
                     axis=1, keepdims=True)
    te = jnp.where(tio[:, 0:1] < total, raw, lastte)
    te_ref[...] = te.astype(jnp.int32)


def _router(x2, Wr):
    return pl.pallas_call(
        _router_body,
        out_shape=[
            jax.ShapeDtypeStruct((NTOK, 1), jnp.float32),
            jax.ShapeDtypeStruct((2 * NTOK, 1), jnp.int32),
            jax.ShapeDtypeStruct((NT, 1), jnp.int32),
            jax.ShapeDtypeStruct((1, 1), jnp.int32),
        ],
        scratch_shapes=[
            pltpu.VMEM((2 * NTOK, NE), jnp.float32),
            pltpu.VMEM((2 * NTOK, 1), jnp.float32),
        ],
    )(x2, Wr)


GNB = 3
GROWS = SEG // NW
GNCH = GROWS // GCH


def _gather_body(x_hbm, rt_hbm, out_hbm, idx_v, b0, b1, b2, g0, g1, g2,
                 s0, s1, s2):
    wid = lax.axis_index("s") * 2 + lax.axis_index("c")
    base = wid * GROWS
    bufs = (b0, b1, b2)
    gsem = (g0, g1, g2)
    wsem = (s0, s1, s2)
    pltpu.sync_copy(rt_hbm.at[pl.ds(base, GROWS)], idx_v)

    gd, wd = {}, {}

    def start_gather(c):
        b = c % GNB
        gd[c] = pltpu.async_copy(
            x_hbm.at[idx_v.at[pl.ds(c * GCH, GCH)]], bufs[b], gsem[b])

    def start_write(c):
        b = c % GNB
        wd[c] = pltpu.async_copy(
            bufs[b], out_hbm.at[pl.ds(base + c * GCH, GCH)], wsem[b])

    for c in range(GNCH):
        if c >= GNB:
            wd[c - GNB].wait()
        start_gather(c)
        if c >= 1:
            gd[c - 1].wait()
            start_write(c - 1)
    gd[GNCH - 1].wait()
    start_write(GNCH - 1)
    for c in range(max(GNCH - GNB, 0), GNCH):
        wd[c].wait()


def _gather(x2, row_token_seg):
    f = functools.partial(
        pl.kernel,
        mesh=plsc.VectorSubcoreMesh(core_axis_name="c", subcore_axis_name="s"),
        out_type=jax.ShapeDtypeStruct((SEG, D), jnp.float32),
        scratch_types=[
            pltpu.VMEM((GROWS,), jnp.int32),
            pltpu.VMEM((GCH, D), jnp.float32),
            pltpu.VMEM((GCH, D), jnp.float32),
            pltpu.VMEM((GCH, D), jnp.float32),
            pltpu.SemaphoreType.DMA,
            pltpu.SemaphoreType.DMA,
            pltpu.SemaphoreType.DMA,
            pltpu.SemaphoreType.DMA,
            pltpu.SemaphoreType.DMA,
            pltpu.SemaphoreType.DMA,
        ],
    )(_gather_body)
    return f(x2, row_token_seg)


_RSQRT2 = 0.7071067811865476


def _ffn_body(te_ref, nt_ref, xs_ref, w1_ref, b1_ref, w2_ref, b2_ref,
              ys_ref):
    j = pl.program_id(0)

    @pl.when(j < nt_ref[0])
    def _():
        xb = xs_ref[...]
        h = lax.dot_general(xb, w1_ref[0], (((1,), (1,)), ((), ())),
                            preferred_element_type=jnp.float32)
        h = h + b1_ref[0]
        h = 0.5 * h * (1.0 + lax.erf(h * _RSQRT2))
        y = lax.dot_general(h, w2_ref[0], (((1,), (1,)), ((), ())),
                            preferred_element_type=jnp.float32)
        ys_ref[...] = y + b2_ref[0]


def _ffn(tile_expert, nact, xs_full, W1, b1r, W2, b2r):
    def _jm(j, nt):
        return jnp.maximum(jnp.minimum(j, nt[0] - 1), 0)

    grid_spec = pltpu.PrefetchScalarGridSpec(
        num_scalar_prefetch=2,
        grid=(NT,),
        in_specs=[
            pl.BlockSpec((TM, D), lambda j, te, nt: (_jm(j, nt), 0)),
            pl.BlockSpec((1, D, D),
                         lambda j, te, nt: (te[_jm(j, nt)], 0, 0)),
            pl.BlockSpec((1, 1, D),
                         lambda j, te, nt: (te[_jm(j, nt)], 0, 0)),
            pl.BlockSpec((1, D, D),
                         lambda j, te, nt: (te[_jm(j, nt)], 0, 0)),
            pl.BlockSpec((1, 1, D),
                         lambda j, te, nt: (te[_jm(j, nt)], 0, 0)),
        ],
        out_specs=pl.BlockSpec((TM, D), lambda j, te, nt: (_jm(j, nt), 0)),
    )
    return pl.pallas_call(
        _ffn_body,
        grid_spec=grid_spec,
        out_shape=jax.ShapeDtypeStruct((NPAD, D), jnp.float32),
        compiler_params=pltpu.CompilerParams(
            dimension_semantics=("arbitrary",)),
    )(tile_expert, nact, xs_full, W1, b1r, W2, b2r)


def _combine_body(ys_hbm, pa_hbm, pb_hbm, wa_hbm, out_hbm,
                  ia_v, ib_v, wa_v, ba_v, bb_v, sa, sb):
    wid = lax.axis_index("s") * 2 + lax.axis_index("c")
    base = wid * (NTOK // NW)

    def chunk(c, carry):
        off = base + c * CCH
        pltpu.sync_copy(pa_hbm.at[pl.ds(off, CCH)], ia_v)
        pltpu.sync_copy(pb_hbm.at[pl.ds(off, CCH)], ib_v)
        pltpu.sync_copy(wa_hbm.at[pl.ds(off, CCH)], wa_v)
        cpa = pltpu.async_copy(ys_hbm.at[ia_v], ba_v, sa)
        cpb = pltpu.async_copy(ys_hbm.at[ib_v], bb_v, sb)
        cpa.wait()
        cpb.wait()

        def row(r, carry2):
            wa = wa_v[r, :]
            for i in range(D // 16):
                sl = pl.ds(i * 16, 16)
                b = bb_v[r, sl]
                ba_v[r, sl] = b + (ba_v[r, sl] - b) * wa
            return carry2

        lax.fori_loop(0, CCH, row, 0)
        pltpu.sync_copy(ba_v, out_hbm.at[pl.ds(off, CCH)])
        return carry

    lax.fori_loop(0, (NTOK // NW) // CCH, chunk, 0)


def _combine(ys, posA, posB, wA):
    f = functools.partial(
        pl.kernel,
        mesh=plsc.VectorSubcoreMesh(core_axis_name="c", subcore_axis_name="s"),
        out_type=jax.ShapeDtypeStruct((NTOK, D), jnp.float32),
        scratch_types=[
            pltpu.VMEM((CCH,), jnp.int32),
            pltpu.VMEM((CCH,), jnp.int32),
            pltpu.VMEM((CCH, 16), jnp.float32),
            pltpu.VMEM((CCH, D), jnp.float32),
            pltpu.VMEM((CCH, D), jnp.float32),
            pltpu.SemaphoreType.DMA,
            pltpu.SemaphoreType.DMA,
        ],
    )(_combine_body)
    return f(ys, posA, posB, wA)


def kernel(x, Wr, W1, b1, W2, b2):
    Bx, L, Dx = x.shape
    x2 = x.reshape(L, Dx)
    wa, pos, te, na = _router(x2, Wr)
    pos_f = pos[:, 0]
    token_ids = jnp.tile(jnp.arange(NTOK, dtype=jnp.int32), 2)
    row_token = jnp.zeros((NPAD,), jnp.int32).at[pos_f].set(token_ids)

    xs = _gather(x2, row_token)
    b1r = b1.reshape(NE, 1, D)
    b2r = b2.reshape(NE, 1, D)
    ys = _ffn(te[:, 0], na[0], xs, W1, b1r, W2, b2r)

    wA16 = jnp.broadcast_to(wa, (NTOK, 16))
    out = _combine(ys, pos_f[:NTOK], pos_f[NTOK:], wA16)
    return out.reshape(Bx, L, Dx)
